# baseline (speedup 1.0000x reference)
import functools

import jax
import jax.numpy as jnp
from jax import lax
from jax.experimental import pallas as pl
from jax.experimental.pallas import tpu as pltpu
from jax.experimental.pallas import tpu_sc as plsc

D = 64
SEQ = 200
BATCH = 4096
SPLIT = 104


def _make_kernel(num_workers):
    seqs_per_w = BATCH // num_workers
    mesh = plsc.VectorSubcoreMesh(core_axis_name="c", subcore_axis_name="s")

    @functools.partial(
        pl.kernel,
        out_type=jax.ShapeDtypeStruct((BATCH, SEQ, D), jnp.float32),
        mesh=mesh,
        scratch_types=[
            pltpu.VMEM((SEQ, D), jnp.float32),
            pltpu.VMEM((SEQ,), jnp.int32),
            pltpu.VMEM((SEQ, D), jnp.float32),
            pltpu.SemaphoreType.DMA,
        ],
        compiler_params=pltpu.CompilerParams(use_tc_tiling_on_sc=False),
    )
    def body(idx_hbm, table_hbm, pos_hbm, out_hbm, pos_v, idx_v, tok_v, sem):
        nc = 2
        wid = lax.axis_index("s") * nc + lax.axis_index("c")
        base = wid * seqs_per_w
        pltpu.sync_copy(pos_hbm, pos_v)

        def per_seq(k, carry):
            b = base + k
            pltpu.sync_copy(idx_hbm.at[b], idx_v)
            pltpu.async_copy(
                table_hbm.at[idx_v.at[pl.ds(0, SPLIT)]],
                tok_v.at[pl.ds(0, SPLIT)], sem)
            pltpu.async_copy(
                table_hbm.at[idx_v.at[pl.ds(SPLIT, SEQ - SPLIT)]],
                tok_v.at[pl.ds(SPLIT, SEQ - SPLIT)], sem)
            pltpu.make_async_copy(
                table_hbm.at[idx_v.at[pl.ds(0, SPLIT)]],
                tok_v.at[pl.ds(0, SPLIT)], sem).wait()
            pltpu.make_async_copy(
                table_hbm.at[idx_v.at[pl.ds(SPLIT, SEQ - SPLIT)]],
                tok_v.at[pl.ds(SPLIT, SEQ - SPLIT)], sem).wait()

            def add_row(r, c):
                for j in range(D // 16):
                    sl = pl.ds(j * 16, 16)
                    tok_v[r, sl] = tok_v[r, sl] + pos_v[r, sl]
                return c

            lax.fori_loop(0, SEQ, add_row, 0, unroll=2)
            pltpu.sync_copy(tok_v, out_hbm.at[b])
            return carry

        lax.fori_loop(0, seqs_per_w, per_seq, 0)

    return body


def kernel(inputs, token_table, pos_table):
    info = plsc.get_sparse_core_info()
    nw = info.num_cores * info.num_subcores
    return _make_kernel(nw)(inputs, token_table, pos_table)
